# baseline (speedup 1.0000x reference)
import jax
import jax.numpy as jnp
from jax import lax
from jax.experimental import pallas as pl
from jax.experimental.pallas import tpu as pltpu
from jax.experimental.pallas import tpu_sc as plsc

SEQ = 200
SEQP = 208
D = 64
BATCH = 4096
VEC = 16
NCORES = 2
NSUB = 16
NW = NCORES * NSUB
SPW = BATCH // NW
SPLIT = (128, 72)
NFULL = SEQ // VEC
TAIL = SEQ - NFULL * VEC


def _sc_body(inputs_hbm, base2_hbm, pos_hbm, out_hbm,
             idxr_v, pos_v, ih0, ih1, g0, g1, o0, o1,
             gs0, gs1, ss0, ss1):
    ihbuf = (ih0, ih1)
    gbuf = (g0, g1)
    obuf = (o0, o1)
    gsem = (gs0, gs1)
    ssem = (ss0, ss1)

    wid = lax.axis_index("c") * NSUB + lax.axis_index("s")
    s0 = pl.multiple_of(wid * SPW, SPW)

    pltpu.sync_copy(inputs_hbm.at[pl.ds(s0 * SEQP, SPW * SEQP)], idxr_v)
    pltpu.sync_copy(pos_hbm, pos_v)

    def build_pair_idx(i, v):
        for t in range(SEQP // VEC):
            ihbuf[v][pl.ds(t * VEC, VEC)] = lax.shift_right_logical(
                idxr_v[pl.ds(i * SEQP + t * VEC, VEC)], 1)

    def start_gather(v):
        off = 0
        for n in SPLIT:
            pltpu.async_copy(base2_hbm.at[ihbuf[v].at[pl.ds(off, n)]],
                             gbuf[v].at[pl.ds(off, n)], gsem[v])
            off += n

    def wait_gather(v):
        off = 0
        for n in SPLIT:
            pltpu.make_async_copy(base2_hbm.at[pl.ds(0, n)],
                                  gbuf[v].at[pl.ds(off, n)], gsem[v]).wait()
            off += n

    def wait_scatter(v):
        pltpu.make_async_copy(obuf[v], out_hbm.at[0], ssem[v]).wait()

    def chunk(i, u, r0, nrows):
        parv = jnp.bitwise_and(idxr_v[pl.ds(i * SEQP + r0, VEC)], 1)
        for j in range(nrows):
            r = r0 + j
            off = lax.shift_left(parv[j], 6)
            for c in range(D // VEC):
                v = gbuf[u][r, pl.ds(off + c * VEC, VEC)]
                p = pos_v[pl.ds(r * D + c * VEC, VEC)]
                obuf[u][pl.ds(r * D + c * VEC, VEC)] = v + p

    def compute(i, u):
        def k_body(k, carry):
            chunk(i, u, k * VEC, VEC)
            return carry
        lax.fori_loop(0, NFULL, k_body, 0)
        chunk(i, u, NFULL * VEC, TAIL)

    build_pair_idx(0, 0)
    start_gather(0)

    def t_body(t, carry):
        for u in range(2):
            i = t * 2 + u

            @pl.when(i + 1 < SPW)
            def _():
                build_pair_idx(i + 1, 1 - u)
                start_gather(1 - u)

            wait_gather(u)

            @pl.when(i >= 2)
            def _():
                wait_scatter(u)

            compute(i, u)
            pltpu.async_copy(obuf[u], out_hbm.at[s0 + i], ssem[u])
        return carry

    lax.fori_loop(0, SPW // 2, t_body, 0)

    for v in range(2):
        wait_scatter(v)


@jax.jit
def _run(inputs_p, base2, pos_f):
    mesh = plsc.VectorSubcoreMesh(core_axis_name="c", subcore_axis_name="s")
    f = pl.kernel(
        _sc_body,
        out_type=jax.ShapeDtypeStruct((BATCH, SEQ * D), jnp.float32),
        mesh=mesh,
        scratch_types=[
            pltpu.VMEM((SPW * SEQP,), jnp.int32),
            pltpu.VMEM((SEQ * D,), jnp.float32),
            *[pltpu.VMEM((SEQP,), jnp.int32) for _ in range(2)],
            *[pltpu.VMEM((SEQ, 2 * D), jnp.float32) for _ in range(2)],
            *[pltpu.VMEM((SEQ * D,), jnp.float32) for _ in range(2)],
            *[pltpu.SemaphoreType.DMA for _ in range(4)],
        ],
    )
    return f(inputs_p, base2, pos_f)


def kernel(inputs, base_table, pos_table):
    inputs_p = jnp.pad(inputs.astype(jnp.int32),
                       ((0, 0), (0, SEQP - SEQ))).reshape(-1)
    base2 = base_table.reshape(500000, 2 * D)
    out = _run(inputs_p, base2, pos_table.reshape(-1))
    return out.reshape(BATCH, SEQ, D)
